# baseline (speedup 1.0000x reference)
import functools

import jax
import jax.numpy as jnp
from jax import lax
from jax.experimental import pallas as pl
from jax.experimental.pallas import tpu as pltpu
from jax.experimental.pallas import tpu_sc as plsc

S = 256
H = S // 2
Y = 1000
B = 16384


YBLK = 40
NG = Y // YBLK


def _k1_body(hbm_ref, o_ref, buf, sems):
    g = pl.program_id(0)
    slot = lax.rem(g, 2)
    nxt = 1 - slot

    def start_group(grp, sl):
        for k in range(YBLK):
            pltpu.make_async_copy(
                hbm_ref.at[:, grp * YBLK + k, :], buf.at[sl, k], sems.at[sl]
            ).start()

    @pl.when(g == 0)
    def _():
        start_group(0, slot)

    @pl.when(g + 1 < NG)
    def _():
        start_group(g + 1, nxt)

    for k in range(YBLK):
        pltpu.make_async_copy(
            hbm_ref.at[:, g * YBLK + k, :], buf.at[slot, k], sems.at[slot]
        ).wait()
    for k in range(YBLK):
        x = buf[slot, k]
        m = jnp.max(x, axis=-1, keepdims=True)
        xs = x - m
        lse = jnp.log(jnp.sum(jnp.exp(xs), axis=-1, keepdims=True))
        t = jnp.transpose(xs - lse)
        o_ref[k, 0:S, :] = t[:, 0:H]
        o_ref[k, S:2 * S, :] = t[:, H:S]


def _k1(T_logits):
    return pl.pallas_call(
        _k1_body,
        grid=(NG,),
        in_specs=[pl.BlockSpec(memory_space=pl.ANY)],
        out_specs=pl.BlockSpec((YBLK, 2 * S, H), lambda i: (i, 0, 0)),
        out_shape=jax.ShapeDtypeStruct((Y, 2 * S, H), jnp.float32),
        scratch_shapes=[
            pltpu.VMEM((2, YBLK, S, S), jnp.float32),
            pltpu.SemaphoreType.DMA((2,)),
        ],
    )(T_logits)


NC, NS = 2, 16
NW = NC * NS
BPW = B // NW
CHUNK = 128
NCHUNK = BPW // CHUNK


def _k2(table, idx4):
    mesh = plsc.VectorSubcoreMesh(core_axis_name="c", subcore_axis_name="s")

    @functools.partial(
        pl.kernel,
        mesh=mesh,
        out_type=jax.ShapeDtypeStruct((2, B, H), jnp.float32),
        scratch_types=[
            pltpu.VMEM((2, NCHUNK, CHUNK), jnp.int32),
            pltpu.VMEM((2, CHUNK, H), jnp.float32),
            pltpu.SemaphoreType.DMA,
            pltpu.SemaphoreType.DMA,
        ],
    )
    def gather_kernel(table_hbm, idx_hbm, out_hbm, idx_v, rows_v, sem0, sem1):
        wid = lax.axis_index("s") * NC + lax.axis_index("c")
        base = wid * BPW
        pltpu.sync_copy(idx_hbm.at[wid], idx_v)
        sems = (sem0, sem1)
        pairs = [(h, c) for h in range(2) for c in range(NCHUNK)]
        pltpu.async_copy(table_hbm.at[idx_v.at[0, 0]], rows_v.at[0], sems[0])
        for p, (h, c) in enumerate(pairs):
            if p + 1 < len(pairs):
                hn, cn = pairs[p + 1]
                pltpu.async_copy(
                    table_hbm.at[idx_v.at[hn, cn]], rows_v.at[(p + 1) % 2],
                    sems[(p + 1) % 2])
            pltpu.make_async_copy(
                table_hbm.at[idx_v.at[h, c]], rows_v.at[p % 2], sems[p % 2]
            ).wait()
            pltpu.sync_copy(
                rows_v.at[p % 2], out_hbm.at[h, pl.ds(base + c * CHUNK, CHUNK)])

    return gather_kernel(table, idx4)


TBLK = 2048
NT = B // TBLK


def _k3_body(hbm_ref, o_ref, buf, sems):
    g = pl.program_id(0)
    slot = lax.rem(g, 2)
    nxt = 1 - slot

    def start_group(grp, sl):
        for h in range(2):
            pltpu.make_async_copy(
                hbm_ref.at[h, pl.ds(grp * TBLK, TBLK), :],
                buf.at[sl, h], sems.at[sl],
            ).start()

    @pl.when(g == 0)
    def _():
        start_group(0, slot)

    @pl.when(g + 1 < NT)
    def _():
        start_group(g + 1, nxt)

    for h in range(2):
        pltpu.make_async_copy(
            hbm_ref.at[h, pl.ds(g * TBLK, TBLK), :], buf.at[slot, h],
            sems.at[slot],
        ).wait()
    o_ref[0:H, :] = jnp.transpose(buf[slot, 0])
    o_ref[H:S, :] = jnp.transpose(buf[slot, 1])


def _k3(out_T):
    return pl.pallas_call(
        _k3_body,
        grid=(NT,),
        in_specs=[pl.BlockSpec(memory_space=pl.ANY)],
        out_specs=pl.BlockSpec((S, TBLK), lambda i: (0, i)),
        out_shape=jax.ShapeDtypeStruct((S, B), jnp.float32),
        scratch_shapes=[
            pltpu.VMEM((2, 2, TBLK, H), jnp.float32),
            pltpu.SemaphoreType.DMA((2,)),
        ],
    )(out_T)


@jax.jit
def kernel(T_logits, symbol_idx, state_idx):
    table = _k1(T_logits).reshape(Y * 2 * S, H)
    y = symbol_idx.astype(jnp.int32)
    j = state_idx.astype(jnp.int32)
    rA = y * (2 * S) + j
    rB = rA + S
    idx4 = jnp.stack(
        [rA.reshape(NW, NCHUNK, CHUNK), rB.reshape(NW, NCHUNK, CHUNK)], axis=1)
    out_T = _k2(table, idx4)
    return _k3(out_T)


# speedup vs baseline: 1.0107x; 1.0107x over previous
import functools

import jax
import jax.numpy as jnp
from jax import lax
from jax.experimental import pallas as pl
from jax.experimental.pallas import tpu as pltpu
from jax.experimental.pallas import tpu_sc as plsc

S = 256
H = S // 2
Y = 1000
B = 16384


YBLK = 40
NG = Y // YBLK


def _k1_body(hbm_ref, o_ref, buf, sems):
    g = pl.program_id(0)
    slot = lax.rem(g, 2)
    nxt = 1 - slot

    def start_group(grp, sl):
        for k in range(YBLK):
            pltpu.make_async_copy(
                hbm_ref.at[:, grp * YBLK + k, :], buf.at[sl, k], sems.at[sl]
            ).start()

    @pl.when(g == 0)
    def _():
        start_group(0, slot)

    @pl.when(g + 1 < NG)
    def _():
        start_group(g + 1, nxt)

    for k in range(YBLK):
        pltpu.make_async_copy(
            hbm_ref.at[:, g * YBLK + k, :], buf.at[slot, k], sems.at[slot]
        ).wait()
    for k in range(YBLK):
        x = buf[slot, k]
        m = jnp.max(x, axis=-1, keepdims=True)
        xs = x - m
        lse = jnp.log(jnp.sum(jnp.exp(xs), axis=-1, keepdims=True))
        t = jnp.transpose(xs - lse)
        o_ref[k, 0:S, :] = t[:, 0:H]
        o_ref[k, S:2 * S, :] = t[:, H:S]


def _k1(T_logits):
    return pl.pallas_call(
        _k1_body,
        grid=(NG,),
        in_specs=[pl.BlockSpec(memory_space=pl.ANY)],
        out_specs=pl.BlockSpec((YBLK, 2 * S, H), lambda i: (i, 0, 0)),
        out_shape=jax.ShapeDtypeStruct((Y, 2 * S, H), jnp.float32),
        scratch_shapes=[
            pltpu.VMEM((2, YBLK, S, S), jnp.float32),
            pltpu.SemaphoreType.DMA((2,)),
        ],
    )(T_logits)


NC, NS = 2, 16
NW = NC * NS
BPW = B // NW
CHUNK = 128
NCHUNK = BPW // CHUNK


def _k2(table, idx4):
    mesh = plsc.VectorSubcoreMesh(core_axis_name="c", subcore_axis_name="s")

    @functools.partial(
        pl.kernel,
        mesh=mesh,
        out_type=jax.ShapeDtypeStruct((2, B, H), jnp.float32),
        scratch_types=[
            pltpu.VMEM((2, NCHUNK, CHUNK), jnp.int32),
            pltpu.VMEM((2, CHUNK, H), jnp.float32),
            pltpu.SemaphoreType.DMA,
            pltpu.SemaphoreType.DMA,
        ],
    )
    def gather_kernel(table_hbm, idx_hbm, out_hbm, idx_v, rows_v, sem0, sem1):
        wid = lax.axis_index("s") * NC + lax.axis_index("c")
        base = wid * BPW
        pltpu.sync_copy(idx_hbm.at[wid], idx_v)
        sems = (sem0, sem1)
        pairs = [(h, c) for h in range(2) for c in range(NCHUNK)]
        pltpu.async_copy(table_hbm.at[idx_v.at[0, 0]], rows_v.at[0], sems[0])
        for p, (h, c) in enumerate(pairs):
            if p + 1 < len(pairs):
                hn, cn = pairs[p + 1]
                pltpu.async_copy(
                    table_hbm.at[idx_v.at[hn, cn]], rows_v.at[(p + 1) % 2],
                    sems[(p + 1) % 2])
            pltpu.make_async_copy(
                table_hbm.at[idx_v.at[h, c]], rows_v.at[p % 2], sems[p % 2]
            ).wait()
            pltpu.sync_copy(
                rows_v.at[p % 2], out_hbm.at[h, pl.ds(base + c * CHUNK, CHUNK)])

    return gather_kernel(table, idx4)


TBLK = 4096


def _k3_body(x_ref, o_ref):
    o_ref[0:H, :] = jnp.transpose(x_ref[0])
    o_ref[H:S, :] = jnp.transpose(x_ref[1])


def _k3(out_T):
    return pl.pallas_call(
        _k3_body,
        grid=(B // TBLK,),
        in_specs=[pl.BlockSpec((2, TBLK, H), lambda i: (0, i, 0))],
        out_specs=pl.BlockSpec((S, TBLK), lambda i: (0, i)),
        out_shape=jax.ShapeDtypeStruct((S, B), jnp.float32),
    )(out_T)


@jax.jit
def kernel(T_logits, symbol_idx, state_idx):
    table = _k1(T_logits).reshape(Y * 2 * S, H)
    y = symbol_idx.astype(jnp.int32)
    j = state_idx.astype(jnp.int32)
    rA = y * (2 * S) + j
    rB = rA + S
    idx4 = jnp.stack(
        [rA.reshape(NW, NCHUNK, CHUNK), rB.reshape(NW, NCHUNK, CHUNK)], axis=1)
    out_T = _k2(table, idx4)
    return _k3(out_T)
